# baseline (speedup 1.0000x reference)
import functools

import jax
import jax.numpy as jnp
from jax import lax
from jax.experimental import pallas as pl
from jax.experimental.pallas import tpu as pltpu
from jax.experimental.pallas import tpu_sc as plsc

BATCH = 16384
EMBED_DIM = 64
PAIR_DIM = 2 * EMBED_DIM
CHUNK = 256


def _sc_gather_pair(user_half_idx, item_half_idx, user_pairs, item_pairs):
    mesh = plsc.VectorSubcoreMesh(core_axis_name="c", subcore_axis_name="s")
    num_workers = mesh.num_cores * mesh.num_subcores
    b_per_w = BATCH // num_workers
    n_chunks = b_per_w // CHUNK

    out_sds = jax.ShapeDtypeStruct((BATCH, PAIR_DIM), jnp.float32)

    @functools.partial(
        pl.kernel,
        mesh=mesh,
        out_type=[out_sds, out_sds],
        scratch_types=[
            pltpu.VMEM((CHUNK,), jnp.int32),
            pltpu.VMEM((CHUNK,), jnp.int32),
            pltpu.VMEM((CHUNK, PAIR_DIM), jnp.float32),
            pltpu.VMEM((CHUNK, PAIR_DIM), jnp.float32),
            pltpu.SemaphoreType.DMA,
            pltpu.SemaphoreType.DMA,
        ],
    )
    def gather_kernel(
        ut_hbm, it_hbm, ui_hbm, ii_hbm,
        ou_hbm, oi_hbm,
        ui_v, ii_v, ur_v, ir_v, sem_u, sem_i,
    ):
        wid = lax.axis_index("s") * mesh.num_cores + lax.axis_index("c")

        @pl.loop(0, n_chunks)
        def _(ci):
            base = wid * b_per_w + ci * CHUNK
            pltpu.sync_copy(ui_hbm.at[pl.ds(base, CHUNK)], ui_v)
            pltpu.sync_copy(ii_hbm.at[pl.ds(base, CHUNK)], ii_v)
            cu = pltpu.async_copy(ut_hbm.at[ui_v], ur_v, sem_u)
            ci_ = pltpu.async_copy(it_hbm.at[ii_v], ir_v, sem_i)
            cu.wait()
            ci_.wait()
            pltpu.sync_copy(ur_v, ou_hbm.at[pl.ds(base, CHUNK)])
            pltpu.sync_copy(ir_v, oi_hbm.at[pl.ds(base, CHUNK)])

    return gather_kernel(user_pairs, item_pairs, user_half_idx, item_half_idx)


def _tc_mlp(user_rows, item_rows, user_par, item_par, W1, b1, W2, b2):
    block = 2048
    grid = BATCH // block

    def body(u_ref, i_ref, up_ref, ip_ref, w1_ref, b1_ref, w2_ref, b2_ref,
             o_ref):
        u = jnp.where(up_ref[...] > 0, u_ref[:, EMBED_DIM:], u_ref[:, :EMBED_DIM])
        i = jnp.where(ip_ref[...] > 0, i_ref[:, EMBED_DIM:], i_ref[:, :EMBED_DIM])
        c = u * i
        h = jnp.dot(c, w1_ref[...], preferred_element_type=jnp.float32)
        h = jnp.maximum(h + b1_ref[...], 0.0)
        o_ref[...] = (
            jnp.dot(h, w2_ref[...], preferred_element_type=jnp.float32)
            + b2_ref[...]
        )

    return pl.pallas_call(
        body,
        grid=(grid,),
        in_specs=[
            pl.BlockSpec((block, PAIR_DIM), lambda i: (i, 0)),
            pl.BlockSpec((block, PAIR_DIM), lambda i: (i, 0)),
            pl.BlockSpec((block, 1), lambda i: (i, 0)),
            pl.BlockSpec((block, 1), lambda i: (i, 0)),
            pl.BlockSpec((EMBED_DIM, EMBED_DIM), lambda i: (0, 0)),
            pl.BlockSpec((1, EMBED_DIM), lambda i: (0, 0)),
            pl.BlockSpec((EMBED_DIM, 1), lambda i: (0, 0)),
            pl.BlockSpec((1, 1), lambda i: (0, 0)),
        ],
        out_specs=pl.BlockSpec((block, 1), lambda i: (i, 0)),
        out_shape=jax.ShapeDtypeStruct((BATCH, 1), jnp.float32),
    )(user_rows, item_rows, user_par, item_par,
      W1, b1.reshape(1, EMBED_DIM), W2, b2.reshape(1, 1))


@jax.jit
def kernel(user_idx, item_idx, user_table, item_table, W1, b1, W2, b2):
    user_idx = user_idx.astype(jnp.int32)
    item_idx = item_idx.astype(jnp.int32)

    user_pairs = user_table.reshape(-1, PAIR_DIM)
    item_pairs = item_table.reshape(-1, PAIR_DIM)

    user_rows, item_rows = _sc_gather_pair(
        user_idx >> 1, item_idx >> 1, user_pairs, item_pairs
    )
    user_par = (user_idx & 1).astype(jnp.int32).reshape(BATCH, 1)
    item_par = (item_idx & 1).astype(jnp.int32).reshape(BATCH, 1)

    out = _tc_mlp(user_rows, item_rows, user_par, item_par, W1, b1, W2, b2)
    return out.squeeze(-1)


# speedup vs baseline: 2.2273x; 2.2273x over previous
import functools

import jax
import jax.numpy as jnp
from jax import lax
from jax.experimental import pallas as pl
from jax.experimental.pallas import tpu as pltpu
from jax.experimental.pallas import tpu_sc as plsc

BATCH = 16384
EMBED_DIM = 64
PAIR_DIM = 2 * EMBED_DIM
CHUNK = 256
NUM_ROWS = 1000000
STRIP = 4096


HALF_ROWS = 503808


def _tc_transpose_pack(table_t):
    def body(a_ref, b_ref, o_ref):
        rows = jax.lax.broadcasted_iota(jnp.int32, (EMBED_DIM, EMBED_DIM), 0)
        cols = jax.lax.broadcasted_iota(jnp.int32, (EMBED_DIM, EMBED_DIM), 1)
        eye = (rows == cols).astype(jnp.bfloat16)
        a16 = a_ref[...].astype(jnp.bfloat16)
        b16 = b_ref[...].astype(jnp.bfloat16)
        o_ref[:, :EMBED_DIM] = jax.lax.dot_general(
            a16, eye, (((0,), (0,)), ((), ())),
            preferred_element_type=jnp.float32,
        )
        o_ref[:, EMBED_DIM:] = jax.lax.dot_general(
            b16, eye, (((0,), (0,)), ((), ())),
            preferred_element_type=jnp.float32,
        )

    n_steps = HALF_ROWS // STRIP

    return pl.pallas_call(
        body,
        grid=(n_steps,),
        in_specs=[
            pl.BlockSpec((EMBED_DIM, STRIP), lambda i: (0, i)),
            pl.BlockSpec(
                (EMBED_DIM, STRIP),
                lambda i: (0, jnp.minimum(n_steps + i, NUM_ROWS // STRIP)),
            ),
        ],
        out_specs=pl.BlockSpec((STRIP, PAIR_DIM), lambda i: (i, 0)),
        out_shape=jax.ShapeDtypeStruct((HALF_ROWS, PAIR_DIM), jnp.float32),
        compiler_params=pltpu.CompilerParams(
            dimension_semantics=("parallel",)
        ),
    )(table_t, table_t)


def _sc_gather_pair(user_half_idx, item_half_idx, user_pairs, item_pairs):
    mesh = plsc.VectorSubcoreMesh(core_axis_name="c", subcore_axis_name="s")
    num_workers = mesh.num_cores * mesh.num_subcores
    b_per_w = BATCH // num_workers
    n_chunks = b_per_w // CHUNK

    out_sds = jax.ShapeDtypeStruct((BATCH, PAIR_DIM), jnp.float32)

    @functools.partial(
        pl.kernel,
        mesh=mesh,
        out_type=[out_sds, out_sds],
        scratch_types=[
            pltpu.VMEM((CHUNK,), jnp.int32),
            pltpu.VMEM((CHUNK,), jnp.int32),
            pltpu.VMEM((CHUNK, PAIR_DIM), jnp.float32),
            pltpu.VMEM((CHUNK, PAIR_DIM), jnp.float32),
            pltpu.SemaphoreType.DMA,
            pltpu.SemaphoreType.DMA,
        ],
    )
    def gather_kernel(
        ut_hbm, it_hbm, ui_hbm, ii_hbm,
        ou_hbm, oi_hbm,
        ui_v, ii_v, ur_v, ir_v, sem_u, sem_i,
    ):
        wid = lax.axis_index("s") * mesh.num_cores + lax.axis_index("c")

        @pl.loop(0, n_chunks)
        def _(ci):
            base = wid * b_per_w + ci * CHUNK
            pltpu.sync_copy(ui_hbm.at[pl.ds(base, CHUNK)], ui_v)
            pltpu.sync_copy(ii_hbm.at[pl.ds(base, CHUNK)], ii_v)
            cu = pltpu.async_copy(ut_hbm.at[ui_v], ur_v, sem_u)
            ci_ = pltpu.async_copy(it_hbm.at[ii_v], ir_v, sem_i)
            cu.wait()
            ci_.wait()
            pltpu.sync_copy(ur_v, ou_hbm.at[pl.ds(base, CHUNK)])
            pltpu.sync_copy(ir_v, oi_hbm.at[pl.ds(base, CHUNK)])

    return gather_kernel(user_pairs, item_pairs, user_half_idx, item_half_idx)


def _tc_mlp(user_rows, item_rows, user_par, item_par, W1, b1, W2, b2):
    block = 2048
    grid = BATCH // block

    def body(u_ref, i_ref, up_ref, ip_ref, w1_ref, b1_ref, w2_ref, b2_ref,
             o_ref):
        u = jnp.where(up_ref[...] > 0, u_ref[:, EMBED_DIM:], u_ref[:, :EMBED_DIM])
        i = jnp.where(ip_ref[...] > 0, i_ref[:, EMBED_DIM:], i_ref[:, :EMBED_DIM])
        c = u * i
        h = jnp.dot(c, w1_ref[...], preferred_element_type=jnp.float32)
        h = jnp.maximum(h + b1_ref[...], 0.0)
        o_ref[...] = (
            jnp.dot(h, w2_ref[...], preferred_element_type=jnp.float32)
            + b2_ref[...]
        )

    return pl.pallas_call(
        body,
        grid=(grid,),
        in_specs=[
            pl.BlockSpec((block, PAIR_DIM), lambda i: (i, 0)),
            pl.BlockSpec((block, PAIR_DIM), lambda i: (i, 0)),
            pl.BlockSpec((block, 1), lambda i: (i, 0)),
            pl.BlockSpec((block, 1), lambda i: (i, 0)),
            pl.BlockSpec((EMBED_DIM, EMBED_DIM), lambda i: (0, 0)),
            pl.BlockSpec((1, EMBED_DIM), lambda i: (0, 0)),
            pl.BlockSpec((EMBED_DIM, 1), lambda i: (0, 0)),
            pl.BlockSpec((1, 1), lambda i: (0, 0)),
        ],
        out_specs=pl.BlockSpec((block, 1), lambda i: (i, 0)),
        out_shape=jax.ShapeDtypeStruct((BATCH, 1), jnp.float32),
    )(user_rows, item_rows, user_par, item_par,
      W1, b1.reshape(1, EMBED_DIM), W2, b2.reshape(1, 1))


@jax.jit
def kernel(user_idx, item_idx, user_table, item_table, W1, b1, W2, b2):
    user_idx = user_idx.astype(jnp.int32)
    item_idx = item_idx.astype(jnp.int32)

    user_pairs = _tc_transpose_pack(user_table.T)
    item_pairs = _tc_transpose_pack(item_table.T)

    user_hi = jnp.where(user_idx >= HALF_ROWS, user_idx - HALF_ROWS, user_idx)
    item_hi = jnp.where(item_idx >= HALF_ROWS, item_idx - HALF_ROWS, item_idx)
    user_rows, item_rows = _sc_gather_pair(
        user_hi, item_hi, user_pairs, item_pairs
    )
    user_par = (user_idx >= HALF_ROWS).astype(jnp.int32).reshape(BATCH, 1)
    item_par = (item_idx >= HALF_ROWS).astype(jnp.int32).reshape(BATCH, 1)

    out = _tc_mlp(user_rows, item_rows, user_par, item_par, W1, b1, W2, b2)
    return out.squeeze(-1)


# speedup vs baseline: 2.5721x; 1.1548x over previous
import functools

import jax
import jax.numpy as jnp
from jax import lax
from jax.experimental import pallas as pl
from jax.experimental.pallas import tpu as pltpu
from jax.experimental.pallas import tpu_sc as plsc

BATCH = 16384
EMBED_DIM = 64
PAIR_DIM = 2 * EMBED_DIM
CHUNK = 256
NUM_ROWS = 1000000
STRIP = 8192


HALF_ROWS = 507904


def _tc_transpose_pack(table_t):
    def body(a_ref, b_ref, o_ref):
        rows = jax.lax.broadcasted_iota(jnp.int32, (EMBED_DIM, EMBED_DIM), 0)
        cols = jax.lax.broadcasted_iota(jnp.int32, (EMBED_DIM, EMBED_DIM), 1)
        eye = (rows == cols).astype(jnp.bfloat16)
        a16 = a_ref[...].astype(jnp.bfloat16)
        b16 = b_ref[...].astype(jnp.bfloat16)
        at = jax.lax.dot_general(
            a16, eye, (((0,), (0,)), ((), ())),
            preferred_element_type=jnp.float32,
        )
        bt = jax.lax.dot_general(
            b16, eye, (((0,), (0,)), ((), ())),
            preferred_element_type=jnp.float32,
        )
        o_ref[...] = jax.lax.concatenate([at, bt], 1)

    n_steps = HALF_ROWS // STRIP

    return pl.pallas_call(
        body,
        grid=(n_steps,),
        in_specs=[
            pl.BlockSpec((EMBED_DIM, STRIP), lambda i: (0, i)),
            pl.BlockSpec(
                (EMBED_DIM, STRIP),
                lambda i: (0, jnp.minimum(n_steps + i, NUM_ROWS // STRIP)),
            ),
        ],
        out_specs=pl.BlockSpec((STRIP, PAIR_DIM), lambda i: (i, 0)),
        out_shape=jax.ShapeDtypeStruct((HALF_ROWS, PAIR_DIM), jnp.float32),
        compiler_params=pltpu.CompilerParams(
            dimension_semantics=("parallel",)
        ),
    )(table_t, table_t)


def _sc_gather_pair(user_half_idx, item_half_idx, user_pairs, item_pairs):
    mesh = plsc.VectorSubcoreMesh(core_axis_name="c", subcore_axis_name="s")
    num_workers = mesh.num_cores * mesh.num_subcores
    b_per_w = BATCH // num_workers
    n_chunks = b_per_w // CHUNK

    out_sds = jax.ShapeDtypeStruct((BATCH, PAIR_DIM), jnp.float32)

    @functools.partial(
        pl.kernel,
        mesh=mesh,
        out_type=[out_sds, out_sds],
        scratch_types=[
            pltpu.VMEM((CHUNK,), jnp.int32),
            pltpu.VMEM((CHUNK,), jnp.int32),
            pltpu.VMEM((CHUNK, PAIR_DIM), jnp.float32),
            pltpu.VMEM((CHUNK, PAIR_DIM), jnp.float32),
            pltpu.SemaphoreType.DMA,
            pltpu.SemaphoreType.DMA,
        ],
    )
    def gather_kernel(
        ut_hbm, it_hbm, ui_hbm, ii_hbm,
        ou_hbm, oi_hbm,
        ui_v, ii_v, ur_v, ir_v, sem_u, sem_i,
    ):
        wid = lax.axis_index("s") * mesh.num_cores + lax.axis_index("c")

        @pl.loop(0, n_chunks)
        def _(ci):
            base = wid * b_per_w + ci * CHUNK
            pltpu.sync_copy(ui_hbm.at[pl.ds(base, CHUNK)], ui_v)
            pltpu.sync_copy(ii_hbm.at[pl.ds(base, CHUNK)], ii_v)
            cu = pltpu.async_copy(ut_hbm.at[ui_v], ur_v, sem_u)
            ci_ = pltpu.async_copy(it_hbm.at[ii_v], ir_v, sem_i)
            cu.wait()
            ci_.wait()
            pltpu.sync_copy(ur_v, ou_hbm.at[pl.ds(base, CHUNK)])
            pltpu.sync_copy(ir_v, oi_hbm.at[pl.ds(base, CHUNK)])

    return gather_kernel(user_pairs, item_pairs, user_half_idx, item_half_idx)


def _tc_mlp(user_rows, item_rows, user_par, item_par, W1, b1, W2, b2):
    block = 2048
    grid = BATCH // block

    def body(u_ref, i_ref, up_ref, ip_ref, w1_ref, b1_ref, w2_ref, b2_ref,
             o_ref):
        u = jnp.where(up_ref[...] > 0, u_ref[:, EMBED_DIM:], u_ref[:, :EMBED_DIM])
        i = jnp.where(ip_ref[...] > 0, i_ref[:, EMBED_DIM:], i_ref[:, :EMBED_DIM])
        c = u * i
        h = jnp.dot(c, w1_ref[...], preferred_element_type=jnp.float32)
        h = jnp.maximum(h + b1_ref[...], 0.0)
        o_ref[...] = (
            jnp.dot(h, w2_ref[...], preferred_element_type=jnp.float32)
            + b2_ref[...]
        )

    return pl.pallas_call(
        body,
        grid=(grid,),
        in_specs=[
            pl.BlockSpec((block, PAIR_DIM), lambda i: (i, 0)),
            pl.BlockSpec((block, PAIR_DIM), lambda i: (i, 0)),
            pl.BlockSpec((block, 1), lambda i: (i, 0)),
            pl.BlockSpec((block, 1), lambda i: (i, 0)),
            pl.BlockSpec((EMBED_DIM, EMBED_DIM), lambda i: (0, 0)),
            pl.BlockSpec((1, EMBED_DIM), lambda i: (0, 0)),
            pl.BlockSpec((EMBED_DIM, 1), lambda i: (0, 0)),
            pl.BlockSpec((1, 1), lambda i: (0, 0)),
        ],
        out_specs=pl.BlockSpec((block, 1), lambda i: (i, 0)),
        out_shape=jax.ShapeDtypeStruct((BATCH, 1), jnp.float32),
    )(user_rows, item_rows, user_par, item_par,
      W1, b1.reshape(1, EMBED_DIM), W2, b2.reshape(1, 1))


@jax.jit
def kernel(user_idx, item_idx, user_table, item_table, W1, b1, W2, b2):
    user_idx = user_idx.astype(jnp.int32)
    item_idx = item_idx.astype(jnp.int32)

    user_pairs = _tc_transpose_pack(user_table.T)
    item_pairs = _tc_transpose_pack(item_table.T)

    user_hi = jnp.where(user_idx >= HALF_ROWS, user_idx - HALF_ROWS, user_idx)
    item_hi = jnp.where(item_idx >= HALF_ROWS, item_idx - HALF_ROWS, item_idx)
    user_rows, item_rows = _sc_gather_pair(
        user_hi, item_hi, user_pairs, item_pairs
    )
    user_par = (user_idx >= HALF_ROWS).astype(jnp.int32).reshape(BATCH, 1)
    item_par = (item_idx >= HALF_ROWS).astype(jnp.int32).reshape(BATCH, 1)

    out = _tc_mlp(user_rows, item_rows, user_par, item_par, W1, b1, W2, b2)
    return out.squeeze(-1)


# speedup vs baseline: 2.5836x; 1.0045x over previous
import functools

import jax
import jax.numpy as jnp
from jax import lax
from jax.experimental import pallas as pl
from jax.experimental.pallas import tpu as pltpu
from jax.experimental.pallas import tpu_sc as plsc

BATCH = 16384
EMBED_DIM = 64
PAIR_DIM = 2 * EMBED_DIM
CHUNK = 256
NUM_ROWS = 1000000
STRIP = 8192


HALF_ROWS = 507904


def _tc_transpose_pack(table_t):
    def body(a_ref, b_ref, o_ref):
        rows = jax.lax.broadcasted_iota(jnp.int32, (EMBED_DIM, EMBED_DIM), 0)
        cols = jax.lax.broadcasted_iota(jnp.int32, (EMBED_DIM, EMBED_DIM), 1)
        eye = (rows == cols).astype(jnp.bfloat16)
        a16 = a_ref[...].astype(jnp.bfloat16)
        b16 = b_ref[...].astype(jnp.bfloat16)
        at = jax.lax.dot_general(
            a16, eye, (((0,), (0,)), ((), ())),
            preferred_element_type=jnp.float32,
        )
        bt = jax.lax.dot_general(
            b16, eye, (((0,), (0,)), ((), ())),
            preferred_element_type=jnp.float32,
        )
        o_ref[...] = jax.lax.concatenate([at, bt], 1)

    n_steps = HALF_ROWS // STRIP

    return pl.pallas_call(
        body,
        grid=(n_steps,),
        in_specs=[
            pl.BlockSpec((EMBED_DIM, STRIP), lambda i: (0, i)),
            pl.BlockSpec(
                (EMBED_DIM, STRIP),
                lambda i: (0, jnp.minimum(n_steps + i, NUM_ROWS // STRIP)),
            ),
        ],
        out_specs=pl.BlockSpec((STRIP, PAIR_DIM), lambda i: (i, 0)),
        out_shape=jax.ShapeDtypeStruct((HALF_ROWS, PAIR_DIM), jnp.float32),
        compiler_params=pltpu.CompilerParams(
            dimension_semantics=("parallel",)
        ),
    )(table_t, table_t)


def _sc_gather(half_idx, pairs):
    mesh = plsc.VectorSubcoreMesh(core_axis_name="c", subcore_axis_name="s")
    num_workers = mesh.num_cores * mesh.num_subcores
    b_per_w = BATCH // num_workers
    n_chunks = b_per_w // CHUNK

    @functools.partial(
        pl.kernel,
        mesh=mesh,
        out_type=jax.ShapeDtypeStruct((BATCH, PAIR_DIM), jnp.float32),
        scratch_types=[
            pltpu.VMEM((CHUNK,), jnp.int32),
            pltpu.VMEM((CHUNK, PAIR_DIM), jnp.float32),
            pltpu.SemaphoreType.DMA,
        ],
    )
    def gather_kernel(t_hbm, i_hbm, o_hbm, i_v, r_v, sem):
        wid = lax.axis_index("s") * mesh.num_cores + lax.axis_index("c")

        @pl.loop(0, n_chunks)
        def _(ci):
            base = wid * b_per_w + ci * CHUNK
            pltpu.sync_copy(i_hbm.at[pl.ds(base, CHUNK)], i_v)
            pltpu.async_copy(t_hbm.at[i_v], r_v, sem).wait()
            pltpu.sync_copy(r_v, o_hbm.at[pl.ds(base, CHUNK)])

    return gather_kernel(pairs, half_idx)


def _tc_mlp(user_rows, item_rows, user_par, item_par, W1, b1, W2, b2):
    block = 2048
    grid = BATCH // block

    def body(u_ref, i_ref, up_ref, ip_ref, w1_ref, b1_ref, w2_ref, b2_ref,
             o_ref):
        u = jnp.where(up_ref[...] > 0, u_ref[:, EMBED_DIM:], u_ref[:, :EMBED_DIM])
        i = jnp.where(ip_ref[...] > 0, i_ref[:, EMBED_DIM:], i_ref[:, :EMBED_DIM])
        c = u * i
        h = jnp.dot(c, w1_ref[...], preferred_element_type=jnp.float32)
        h = jnp.maximum(h + b1_ref[...], 0.0)
        o_ref[...] = (
            jnp.dot(h, w2_ref[...], preferred_element_type=jnp.float32)
            + b2_ref[...]
        )

    return pl.pallas_call(
        body,
        grid=(grid,),
        in_specs=[
            pl.BlockSpec((block, PAIR_DIM), lambda i: (i, 0)),
            pl.BlockSpec((block, PAIR_DIM), lambda i: (i, 0)),
            pl.BlockSpec((block, 1), lambda i: (i, 0)),
            pl.BlockSpec((block, 1), lambda i: (i, 0)),
            pl.BlockSpec((EMBED_DIM, EMBED_DIM), lambda i: (0, 0)),
            pl.BlockSpec((1, EMBED_DIM), lambda i: (0, 0)),
            pl.BlockSpec((EMBED_DIM, 1), lambda i: (0, 0)),
            pl.BlockSpec((1, 1), lambda i: (0, 0)),
        ],
        out_specs=pl.BlockSpec((block, 1), lambda i: (i, 0)),
        out_shape=jax.ShapeDtypeStruct((BATCH, 1), jnp.float32),
    )(user_rows, item_rows, user_par, item_par,
      W1, b1.reshape(1, EMBED_DIM), W2, b2.reshape(1, 1))


@jax.jit
def kernel(user_idx, item_idx, user_table, item_table, W1, b1, W2, b2):
    user_idx = user_idx.astype(jnp.int32)
    item_idx = item_idx.astype(jnp.int32)

    user_hi = jnp.where(user_idx >= HALF_ROWS, user_idx - HALF_ROWS, user_idx)
    item_hi = jnp.where(item_idx >= HALF_ROWS, item_idx - HALF_ROWS, item_idx)

    user_pairs = _tc_transpose_pack(user_table.T)
    user_rows = _sc_gather(user_hi, user_pairs)
    item_pairs = _tc_transpose_pack(item_table.T)
    item_rows = _sc_gather(item_hi, item_pairs)
    user_par = (user_idx >= HALF_ROWS).astype(jnp.int32).reshape(BATCH, 1)
    item_par = (item_idx >= HALF_ROWS).astype(jnp.int32).reshape(BATCH, 1)

    out = _tc_mlp(user_rows, item_rows, user_par, item_par, W1, b1, W2, b2)
    return out.squeeze(-1)


# speedup vs baseline: 2.8244x; 1.0932x over previous
import functools

import jax
import jax.numpy as jnp
from jax import lax
from jax.experimental import pallas as pl
from jax.experimental.pallas import tpu as pltpu
from jax.experimental.pallas import tpu_sc as plsc

BATCH = 16384
EMBED_DIM = 64
PAIR_DIM = 2 * EMBED_DIM
CHUNK = 256
NUM_ROWS = 1000000
STRIP = 16384


HALF_ROWS = 507904


def _tc_transpose_pack(table_t):
    def body(a_ref, b_ref, o_ref):
        rows = jax.lax.broadcasted_iota(jnp.int32, (EMBED_DIM, EMBED_DIM), 0)
        cols = jax.lax.broadcasted_iota(jnp.int32, (EMBED_DIM, EMBED_DIM), 1)
        eye = (rows == cols).astype(jnp.bfloat16)
        a16 = a_ref[...].astype(jnp.bfloat16)
        b16 = b_ref[...].astype(jnp.bfloat16)
        at = jax.lax.dot_general(
            a16, eye, (((0,), (0,)), ((), ())),
            preferred_element_type=jnp.float32,
        )
        bt = jax.lax.dot_general(
            b16, eye, (((0,), (0,)), ((), ())),
            preferred_element_type=jnp.float32,
        )
        o_ref[...] = jax.lax.concatenate([at, bt], 1)

    n_steps = HALF_ROWS // STRIP

    return pl.pallas_call(
        body,
        grid=(n_steps,),
        in_specs=[
            pl.BlockSpec((EMBED_DIM, STRIP), lambda i: (0, i)),
            pl.BlockSpec(
                (EMBED_DIM, STRIP),
                lambda i: (0, jnp.minimum(n_steps + i, NUM_ROWS // STRIP)),
            ),
        ],
        out_specs=pl.BlockSpec((STRIP, PAIR_DIM), lambda i: (i, 0)),
        out_shape=jax.ShapeDtypeStruct((HALF_ROWS, PAIR_DIM), jnp.float32),
        compiler_params=pltpu.CompilerParams(
            dimension_semantics=("parallel",)
        ),
    )(table_t, table_t)


def _sc_gather(half_idx, pairs):
    mesh = plsc.VectorSubcoreMesh(core_axis_name="c", subcore_axis_name="s")
    num_workers = mesh.num_cores * mesh.num_subcores
    b_per_w = BATCH // num_workers
    n_chunks = b_per_w // CHUNK

    @functools.partial(
        pl.kernel,
        mesh=mesh,
        out_type=jax.ShapeDtypeStruct((BATCH, PAIR_DIM), jnp.float32),
        scratch_types=[
            pltpu.VMEM((CHUNK,), jnp.int32),
            pltpu.VMEM((CHUNK, PAIR_DIM), jnp.float32),
            pltpu.SemaphoreType.DMA,
        ],
    )
    def gather_kernel(t_hbm, i_hbm, o_hbm, i_v, r_v, sem):
        wid = lax.axis_index("s") * mesh.num_cores + lax.axis_index("c")

        @pl.loop(0, n_chunks)
        def _(ci):
            base = wid * b_per_w + ci * CHUNK
            pltpu.sync_copy(i_hbm.at[pl.ds(base, CHUNK)], i_v)
            pltpu.async_copy(t_hbm.at[i_v], r_v, sem).wait()
            pltpu.sync_copy(r_v, o_hbm.at[pl.ds(base, CHUNK)])

    return gather_kernel(pairs, half_idx)


def _tc_mlp(user_rows, item_rows, user_par, item_par, W1, b1, W2, b2):
    block = 2048
    grid = BATCH // block

    def body(u_ref, i_ref, up_ref, ip_ref, w1_ref, b1_ref, w2_ref, b2_ref,
             o_ref):
        u = jnp.where(up_ref[...] > 0, u_ref[:, EMBED_DIM:], u_ref[:, :EMBED_DIM])
        i = jnp.where(ip_ref[...] > 0, i_ref[:, EMBED_DIM:], i_ref[:, :EMBED_DIM])
        c = u * i
        h = jnp.dot(c, w1_ref[...], preferred_element_type=jnp.float32)
        h = jnp.maximum(h + b1_ref[...], 0.0)
        o_ref[...] = (
            jnp.dot(h, w2_ref[...], preferred_element_type=jnp.float32)
            + b2_ref[...]
        )

    return pl.pallas_call(
        body,
        grid=(grid,),
        in_specs=[
            pl.BlockSpec((block, PAIR_DIM), lambda i: (i, 0)),
            pl.BlockSpec((block, PAIR_DIM), lambda i: (i, 0)),
            pl.BlockSpec((block, 1), lambda i: (i, 0)),
            pl.BlockSpec((block, 1), lambda i: (i, 0)),
            pl.BlockSpec((EMBED_DIM, EMBED_DIM), lambda i: (0, 0)),
            pl.BlockSpec((1, EMBED_DIM), lambda i: (0, 0)),
            pl.BlockSpec((EMBED_DIM, 1), lambda i: (0, 0)),
            pl.BlockSpec((1, 1), lambda i: (0, 0)),
        ],
        out_specs=pl.BlockSpec((block, 1), lambda i: (i, 0)),
        out_shape=jax.ShapeDtypeStruct((BATCH, 1), jnp.float32),
    )(user_rows, item_rows, user_par, item_par,
      W1, b1.reshape(1, EMBED_DIM), W2, b2.reshape(1, 1))


@jax.jit
def kernel(user_idx, item_idx, user_table, item_table, W1, b1, W2, b2):
    user_idx = user_idx.astype(jnp.int32)
    item_idx = item_idx.astype(jnp.int32)

    user_hi = jnp.where(user_idx >= HALF_ROWS, user_idx - HALF_ROWS, user_idx)
    item_hi = jnp.where(item_idx >= HALF_ROWS, item_idx - HALF_ROWS, item_idx)

    user_pairs = _tc_transpose_pack(user_table.T)
    user_rows = _sc_gather(user_hi, user_pairs)
    item_pairs = _tc_transpose_pack(item_table.T)
    item_rows = _sc_gather(item_hi, item_pairs)
    user_par = (user_idx >= HALF_ROWS).astype(jnp.int32).reshape(BATCH, 1)
    item_par = (item_idx >= HALF_ROWS).astype(jnp.int32).reshape(BATCH, 1)

    out = _tc_mlp(user_rows, item_rows, user_par, item_par, W1, b1, W2, b2)
    return out.squeeze(-1)
